# baseline (speedup 1.0000x reference)
import functools

import jax
import jax.numpy as jnp
from jax import lax
from jax.experimental import pallas as pl
from jax.experimental.pallas import tpu as pltpu
from jax.experimental.pallas import tpu_sc as plsc

_N = 100000
_G = 1024
_NC = 2
_NS = 16
_NW = _NC * _NS
_CHUNK = 3136
_TAIL = _N - 31 * _CHUNK
_STEPS = _CHUNK // 16
_TAIL_STEPS = _TAIL // 16
_LANES = 16

_mesh = plsc.VectorSubcoreMesh(core_axis_name="c", subcore_axis_name="s")


@functools.partial(
    pl.kernel,
    mesh=_mesh,
    compiler_params=pltpu.CompilerParams(
        needs_layout_passes=False,
        disable_bounds_checks=True,
        disable_semaphore_checks=True,
    ),
    out_type=jax.ShapeDtypeStruct((_NW, _G), jnp.float32),
    scratch_types=[
        pltpu.VMEM((_CHUNK,), jnp.float32),
        pltpu.VMEM((_CHUNK,), jnp.int32),
        pltpu.VMEM((_G,), jnp.float32),
    ],
)
def _segment_sum_sc(energy_hbm, idx_hbm, out_hbm, e_v, i_v, acc_v):
    wid = lax.axis_index("s") * _NC + lax.axis_index("c")
    base = wid * _CHUNK

    zeros = jnp.zeros((_LANES,), jnp.float32)
    for j in range(_G // _LANES):
        acc_v[pl.ds(j * _LANES, _LANES)] = zeros

    @pl.when(wid < _NW - 1)
    def _():
        pltpu.sync_copy(energy_hbm.at[pl.ds(base, _CHUNK)], e_v)
        pltpu.sync_copy(idx_hbm.at[pl.ds(base, _CHUNK)], i_v)

    @pl.when(wid == _NW - 1)
    def _():
        pltpu.sync_copy(energy_hbm.at[pl.ds(base, _TAIL)], e_v.at[pl.ds(0, _TAIL)])
        pltpu.sync_copy(idx_hbm.at[pl.ds(base, _TAIL)], i_v.at[pl.ds(0, _TAIL)])

    steps = jnp.where(wid == _NW - 1, _TAIL_STEPS, _STEPS)

    def body(j, carry):
        e = e_v[pl.ds(j * _LANES, _LANES)]
        ix = i_v[pl.ds(j * _LANES, _LANES)]
        plsc.addupdate_scatter(acc_v, [ix], e)
        return carry

    lax.fori_loop(0, steps, body, 0)
    pltpu.sync_copy(acc_v, out_hbm.at[wid])


def _reduce_body(x_ref, o_ref):
    o_ref[...] = jnp.sum(x_ref[...], axis=0)


def kernel(node_energy, batch, num_graphs):
    del num_graphs
    partial = _segment_sum_sc(
        node_energy.astype(jnp.float32), batch.astype(jnp.int32)
    )
    return pl.pallas_call(
        _reduce_body,
        out_shape=jax.ShapeDtypeStruct((_G,), jnp.float32),
    )(partial)


# speedup vs baseline: 1.0275x; 1.0275x over previous
import functools

import jax
import jax.numpy as jnp
from jax import lax
from jax.experimental import pallas as pl
from jax.experimental.pallas import tpu as pltpu
from jax.experimental.pallas import tpu_sc as plsc

_N = 100000
_G = 1024
_NC = 2
_NS = 16
_NW = _NC * _NS
_CHUNK = 3136
_TAIL = _N - 31 * _CHUNK
_STEPS = _CHUNK // 16
_TAIL_STEPS = _TAIL // 16
_LANES = 16

_mesh = plsc.VectorSubcoreMesh(core_axis_name="c", subcore_axis_name="s")


@functools.partial(
    pl.kernel,
    mesh=_mesh,
    compiler_params=pltpu.CompilerParams(
        needs_layout_passes=False,
        disable_bounds_checks=True,
        disable_semaphore_checks=True,
    ),
    out_type=jax.ShapeDtypeStruct((_NW, _G), jnp.float32),
    scratch_types=[
        pltpu.VMEM((_CHUNK,), jnp.float32),
        pltpu.VMEM((_CHUNK,), jnp.int32),
        pltpu.VMEM((_G,), jnp.float32),
    ],
)
def _segment_sum_sc(energy_hbm, idx_hbm, out_hbm, e_v, i_v, acc_v):
    wid = lax.axis_index("s") * _NC + lax.axis_index("c")
    base = wid * _CHUNK

    zeros = jnp.zeros((_LANES,), jnp.float32)
    for j in range(_G // _LANES):
        acc_v[pl.ds(j * _LANES, _LANES)] = zeros

    @pl.when(wid < _NW - 1)
    def _():
        pltpu.sync_copy(energy_hbm.at[pl.ds(base, _CHUNK)], e_v)
        pltpu.sync_copy(idx_hbm.at[pl.ds(base, _CHUNK)], i_v)

    @pl.when(wid == _NW - 1)
    def _():
        pltpu.sync_copy(energy_hbm.at[pl.ds(base, _TAIL)], e_v.at[pl.ds(0, _TAIL)])
        pltpu.sync_copy(idx_hbm.at[pl.ds(base, _TAIL)], i_v.at[pl.ds(0, _TAIL)])

    steps = jnp.where(wid == _NW - 1, _TAIL_STEPS, _STEPS)

    @plsc.parallel_loop(0, steps * _LANES, _LANES, unroll=8)
    def _(off):
        e = e_v[pl.ds(off, _LANES)]
        ix = i_v[pl.ds(off, _LANES)]
        plsc.addupdate_scatter(acc_v, [ix], e)
    pltpu.sync_copy(acc_v, out_hbm.at[wid])


def _reduce_body(x_ref, o_ref):
    o_ref[...] = jnp.sum(x_ref[...], axis=0)


def kernel(node_energy, batch, num_graphs):
    del num_graphs
    partial = _segment_sum_sc(
        node_energy.astype(jnp.float32), batch.astype(jnp.int32)
    )
    return pl.pallas_call(
        _reduce_body,
        out_shape=jax.ShapeDtypeStruct((_G,), jnp.float32),
    )(partial)
